# baseline (speedup 1.0000x reference)
import functools

import jax
import jax.numpy as jnp
from jax import lax
from jax.experimental import pallas as pl
from jax.experimental.pallas import tpu as pltpu
from jax.experimental.pallas import tpu_sc as plsc

PAD = 0
D_MODEL = 64
NUM_CORES = 2
NUM_SUBCORES = 16
NUM_WORKERS = NUM_CORES * NUM_SUBCORES

IDX_PW = 6400
DESC = 128
N_DESC = IDX_PW // DESC


def _masks_body(tok_ref, pad_ref, seq_ref):
    pad_ref[...] = tok_ref[...] == PAD
    n = seq_ref.shape[0]
    row = lax.broadcasted_iota(jnp.int32, (n, n), 0)
    col = lax.broadcasted_iota(jnp.int32, (n, n), 1)
    seq_ref[...] = col > row


@functools.lru_cache(maxsize=None)
def _make_gather(n_tok):
    assert n_tok == IDX_PW * NUM_WORKERS
    mesh = plsc.VectorSubcoreMesh(core_axis_name="c", subcore_axis_name="s")

    @functools.partial(
        pl.kernel,
        mesh=mesh,
        out_type=jax.ShapeDtypeStruct((n_tok, D_MODEL), jnp.float32),
        scratch_types=[
            pltpu.VMEM((N_DESC, DESC), jnp.int32),
            pltpu.VMEM((2, DESC, D_MODEL), jnp.float32),
            pltpu.SemaphoreType.DMA,
            pltpu.SemaphoreType.DMA,
        ],
        compiler_params=pltpu.CompilerParams(use_tc_tiling_on_sc=False),
    )
    def k(tok_hbm, table_hbm, out_hbm, idx_v, rows_v, sem0, sem1):
        sems = (sem0, sem1)
        wid = lax.axis_index("s") * NUM_CORES + lax.axis_index("c")
        base = wid * IDX_PW

        pltpu.sync_copy(tok_hbm.at[pl.ds(wid * N_DESC, N_DESC)], idx_v)

        def issue_gather(d, b):
            pltpu.async_copy(table_hbm.at[idx_v.at[d]], rows_v.at[b], sems[b])

        def wait_gather(b):
            pltpu.make_async_copy(table_hbm.at[pl.ds(0, DESC)], rows_v.at[b],
                                  sems[b]).wait()

        issue_gather(0, 0)
        issue_gather(1, 1)

        def body(i, carry):
            for b in range(2):
                d = 2 * i + b
                wait_gather(b)
                pltpu.sync_copy(rows_v.at[b],
                                out_hbm.at[pl.ds(base + d * DESC, DESC)])
                issue_gather(d + 2, b)
            return carry

        lax.fori_loop(0, N_DESC // 2 - 1, body, 0)

        for b in range(2):
            d = N_DESC - 2 + b
            wait_gather(b)
            pltpu.sync_copy(rows_v.at[b],
                            out_hbm.at[pl.ds(base + d * DESC, DESC)])

    return k


def kernel(tokens, table):
    bsz, seq_len = tokens.shape
    tok32 = tokens.astype(jnp.int32)
    feats = _make_gather(bsz * seq_len)(tok32.reshape(-1, DESC), table)
    pad, seq = pl.pallas_call(
        _masks_body,
        out_shape=(
            jax.ShapeDtypeStruct((bsz, seq_len), jnp.bool_),
            jax.ShapeDtypeStruct((seq_len, seq_len), jnp.bool_),
        ),
    )(tok32)
    return (feats.reshape(bsz, seq_len, D_MODEL),
            pad[:, None, None, :], seq)
